# baseline (speedup 1.0000x reference)
import jax
import jax.numpy as jnp
from jax.experimental import pallas as pl
from jax.experimental.pallas import tpu as pltpu

_N = 100000
_D = 128
_BLOCK = 10000
_GRID = _N // _BLOCK // 2
_AW = 40


def _sum_body(x1_ref, x2_ref, o_ref, acc_ref):
    @pl.when(pl.program_id(0) == 0)
    def _():
        acc_ref[...] = jnp.zeros_like(acc_ref)

    acc_ref[...] += (jnp.sum(x1_ref[...].reshape(-1, _AW, _D), axis=0)
                     + jnp.sum(x2_ref[...].reshape(-1, _AW, _D), axis=0))

    @pl.when(pl.program_id(0) == pl.num_programs(0) - 1)
    def _():
        o_ref[...] = jnp.sum(acc_ref[...], axis=0, keepdims=True)


def kernel(x):
    return pl.pallas_call(
        _sum_body,
        grid=(_GRID,),
        in_specs=[
            pl.BlockSpec((_BLOCK, _D), lambda i: (i, 0)),
            pl.BlockSpec((_BLOCK, _D), lambda i: (i + _GRID, 0)),
        ],
        out_specs=pl.BlockSpec((1, _D), lambda i: (0, 0)),
        out_shape=jax.ShapeDtypeStruct((1, _D), jnp.float32),
        scratch_shapes=[pltpu.VMEM((_AW, _D), jnp.float32)],
    )(x, x)


# speedup vs baseline: 1.0296x; 1.0296x over previous
import jax
import jax.numpy as jnp
from jax.experimental import pallas as pl
from jax.experimental.pallas import tpu as pltpu

_N = 100000
_D = 128
_BLOCK = 10000
_AW = 40


def _sum_body(x_ref, o_ref, acc_ref):
    @pl.when(pl.program_id(0) == 0)
    def _():
        acc_ref[...] = jnp.zeros_like(acc_ref)

    acc_ref[...] += jnp.sum(x_ref[...].reshape(-1, _AW, _D), axis=0)

    @pl.when(pl.program_id(0) == pl.num_programs(0) - 1)
    def _():
        o_ref[...] = jnp.sum(acc_ref[...], axis=0, keepdims=True)


def kernel(x):
    return pl.pallas_call(
        _sum_body,
        grid=(_N // _BLOCK,),
        in_specs=[pl.BlockSpec((_BLOCK, _D), lambda i: (i, 0))],
        out_specs=pl.BlockSpec((1, _D), lambda i: (0, 0)),
        out_shape=jax.ShapeDtypeStruct((1, _D), jnp.float32),
        scratch_shapes=[pltpu.VMEM((_AW, _D), jnp.float32)],
    )(x)
